# baseline (speedup 1.0000x reference)
import jax
import jax.numpy as jnp
import numpy as np
from jax import lax
from jax.experimental import pallas as pl
from jax.experimental.pallas import tpu as pltpu
from jax.experimental.pallas import tpu_sc as plsc

F = 128
T = 64
CIN = 64
S = 20
BF = 768
B = 1024
E = 8
H4 = 512
G3 = 384
DFF = 256
DE = 512


def _fullspec(shape):
    nd = len(shape)
    return pl.BlockSpec(shape, lambda i, _n=nd: (0,) * _n)


def _lnorm(x, g, b):
    m = jnp.mean(x, axis=-1, keepdims=True)
    v = jnp.mean((x - m) ** 2, axis=-1, keepdims=True)
    return (x - m) / jnp.sqrt(v + 1e-5) * g + b


def _softmax_last(x):
    m = jnp.max(x, axis=-1, keepdims=True)
    e = jnp.exp(x - m)
    return e / jnp.sum(e, axis=-1, keepdims=True)


def _conv_lstm_body(ts_ref, Wc1_ref, bc1_ref, Wc2_ref, bc2_ref,
                    Wxh_ref, bl_ref, hs_ref, h2_ref):
    BB = ts_ref.shape[0]
    x = ts_ref[...]

    xf = x.reshape(BB * T, CIN)
    z0 = jnp.dot(xf, Wc1_ref[0], preferred_element_type=jnp.float32).reshape(BB, T, F)
    z1 = jnp.dot(xf, Wc1_ref[1], preferred_element_type=jnp.float32).reshape(BB, T, F)
    z2 = jnp.dot(xf, Wc1_ref[2], preferred_element_type=jnp.float32).reshape(BB, T, F)
    y = z2
    y = y + jnp.concatenate([jnp.zeros((BB, 1, F), jnp.float32), z1[:, : T - 1]], axis=1)
    y = y + jnp.concatenate([jnp.zeros((BB, 2, F), jnp.float32), z0[:, : T - 2]], axis=1)
    h1 = jnp.maximum(y + bc1_ref[...], 0.0)

    hf = h1.reshape(BB * T, F)
    z0 = jnp.dot(hf, Wc2_ref[0], preferred_element_type=jnp.float32).reshape(BB, T, F)
    z1 = jnp.dot(hf, Wc2_ref[1], preferred_element_type=jnp.float32).reshape(BB, T, F)
    z2 = jnp.dot(hf, Wc2_ref[2], preferred_element_type=jnp.float32).reshape(BB, T, F)
    y = z2
    y = y + jnp.concatenate([jnp.zeros((BB, 2, F), jnp.float32), z1[:, : T - 2]], axis=1)
    y = y + jnp.concatenate([jnp.zeros((BB, 4, F), jnp.float32), z0[:, : T - 4]], axis=1)
    h2_ref[...] = jnp.maximum(y + bc2_ref[...], 0.0)

    Wxh = Wxh_ref[...]
    bl = bl_ref[...]

    def step(t, carry):
        h, cst = carry
        xt = h2_ref[:, pl.ds(t, 1), :].reshape(BB, F)
        zcat = jnp.concatenate([xt, h], axis=-1)
        zt = jnp.dot(zcat, Wxh, preferred_element_type=jnp.float32) + bl
        ig = jax.nn.sigmoid(zt[:, 0:F])
        fg = jax.nn.sigmoid(zt[:, F:2 * F])
        gg = jnp.tanh(zt[:, 2 * F:3 * F])
        og = jax.nn.sigmoid(zt[:, 3 * F:4 * F])
        cst = fg * cst + ig * gg
        h = og * jnp.tanh(cst)
        hs_ref[:, pl.ds(t, 1), :] = h[:, None, :]
        return (h, cst)

    zero = jnp.zeros((BB, F), jnp.float32)
    lax.fori_loop(0, T, step, (zero, zero))


def _mha_ffn_body(hs_ref, Wq_ref, Wk_ref, Wv_ref, Wo_ref,
                  ln1g_ref, ln1b_ref, Wf1_ref, bf1_ref, Wf2_ref, bf2_ref,
                  ln2g_ref, ln2b_ref, Wts_ref, bts_ref, out_ref):
    BB = hs_ref.shape[0]
    hs = hs_ref[...]
    hflat = hs.reshape(BB * T, F)

    q = jnp.dot(hflat, Wq_ref[...], preferred_element_type=jnp.float32).reshape(BB, T, F)
    k = jnp.dot(hflat, Wk_ref[...], preferred_element_type=jnp.float32).reshape(BB, T, F)
    v = jnp.dot(hflat, Wv_ref[...], preferred_element_type=jnp.float32).reshape(BB, T, F)
    hd = F // 4
    scale = np.float32(1.0 / np.sqrt(hd))
    heads = []
    for hh in range(4):
        sl = slice(hh * hd, (hh + 1) * hd)
        qh = q[:, :, sl]
        kh = k[:, :, sl]
        vh = v[:, :, sl]
        s = lax.dot_general(qh, kh, dimension_numbers=(((2,), (2,)), ((0,), (0,))),
                            preferred_element_type=jnp.float32) * scale
        a = _softmax_last(s)
        oh = lax.dot_general(a, vh, dimension_numbers=(((2,), (1,)), ((0,), (0,))),
                             preferred_element_type=jnp.float32)
        heads.append(oh)
    mh = jnp.concatenate(heads, axis=-1).reshape(BB * T, F)
    mo = jnp.dot(mh, Wo_ref[...], preferred_element_type=jnp.float32)

    h = _lnorm(hflat + mo, ln1g_ref[...], ln1b_ref[...])
    f1 = jnp.maximum(jnp.dot(h, Wf1_ref[...], preferred_element_type=jnp.float32) + bf1_ref[...], 0.0)
    f2 = jnp.dot(f1, Wf2_ref[...], preferred_element_type=jnp.float32) + bf2_ref[...]
    h = _lnorm(h + f2, ln2g_ref[...], ln2b_ref[...])

    pooled = jnp.mean(h.reshape(BB, T, F), axis=1)
    out_ref[...] = jnp.dot(pooled, Wts_ref[...], preferred_element_type=jnp.float32) + bts_ref[...]


def _gru_branch_body(bert_ref, tsf_ref, Wgx_ref, bgr_ref, Wgh_ref,
                     Wp_ref, bp_ref, Wg_ref, bg_ref,
                     x_ref, logit_ref, xl_ref):
    BB = bert_ref.shape[0]
    bx = bert_ref[...].reshape(BB * S, BF)
    xl = jnp.dot(bx, Wgx_ref[...], preferred_element_type=jnp.float32) + bgr_ref[...]
    xl_ref[...] = xl.reshape(BB, S, G3)

    Wgh = Wgh_ref[...]

    def step(t, h):
        xlt = xl_ref[:, pl.ds(t, 1), :].reshape(BB, G3)
        hl = jnp.dot(h, Wgh, preferred_element_type=jnp.float32)
        r = jax.nn.sigmoid(xlt[:, 0:F] + hl[:, 0:F])
        zz = jax.nn.sigmoid(xlt[:, F:2 * F] + hl[:, F:2 * F])
        n = jnp.tanh(xlt[:, 2 * F:3 * F] + r * hl[:, 2 * F:3 * F])
        return (1.0 - zz) * n + zz * h

    h = lax.fori_loop(0, S, step, jnp.zeros((BB, F), jnp.float32))
    text_feat = jnp.dot(h, Wp_ref[...], preferred_element_type=jnp.float32) + bp_ref[...]

    x_ref[:, 0:F] = tsf_ref[...]
    x_ref[:, F:2 * F] = text_feat
    xcat = x_ref[...]
    logit_ref[...] = jnp.dot(xcat, Wg_ref[...], preferred_element_type=jnp.float32) + bg_ref[...]


def _router_sc_body(lt_hbm, w_hbm, lt_v, w_v):
    wid = lax.axis_index("s") * 2 + lax.axis_index("c")
    rows = B // 32
    base = wid * rows * E
    pltpu.sync_copy(lt_hbm.at[pl.ds(base, rows * E)], lt_v)
    for c in range(rows // 16):
        ls = [lt_v[pl.ds(e * rows + c * 16, 16)] for e in range(E)]
        m = ls[0]
        for e in range(1, E):
            m = jnp.maximum(m, ls[e])
        ex = [jnp.exp(l - m) for l in ls]
        ssum = ex[0]
        for e in range(1, E):
            ssum = ssum + ex[e]
        ps = [v / ssum for v in ex]
        v1 = ps[0]
        for e in range(1, E):
            v1 = jnp.maximum(v1, ps[e])
        i1 = jnp.full((16,), E, jnp.int32)
        for e in reversed(range(E)):
            i1 = jnp.where(ps[e] == v1, e, i1)
        p2 = [jnp.where(i1 == e, -1.0, ps[e]) for e in range(E)]
        v2 = p2[0]
        for e in range(1, E):
            v2 = jnp.maximum(v2, p2[e])
        i2 = jnp.full((16,), E, jnp.int32)
        for e in reversed(range(E)):
            i2 = jnp.where(p2[e] == v2, e, i2)
        denom = v1 + v2
        for e in range(E):
            w_v[pl.ds(e * rows + c * 16, 16)] = jnp.where(
                i1 == e, v1, jnp.where(i2 == e, v2, 0.0)) / denom
    pltpu.sync_copy(w_v, w_hbm.at[pl.ds(base, rows * E)])


def _router_sc(logits):
    rows = B // 32
    lt_blk = logits.T.reshape(E, 32, rows).transpose(1, 0, 2).reshape(-1)
    w_blk = pl.kernel(
        _router_sc_body,
        mesh=plsc.VectorSubcoreMesh(core_axis_name="c", subcore_axis_name="s"),
        out_type=jax.ShapeDtypeStruct((B * E,), jnp.float32),
        scratch_types=[
            pltpu.VMEM((rows * E,), jnp.float32),
            pltpu.VMEM((rows * E,), jnp.float32),
        ],
    )(lt_blk)
    return w_blk.reshape(32, E, rows).transpose(1, 0, 2).reshape(E, B).T


def _moe_body(x_ref, w_ref, We1_ref, be1_ref, We2_ref, be2_ref,
              Wt1a_ref, bt1a_ref, Wt1b_ref, bt1b_ref,
              Wt2a_ref, bt2a_ref, Wt2b_ref, bt2b_ref,
              out_ref):
    x = x_ref[...]
    w = w_ref[...]

    moe = jnp.zeros((x.shape[0], 2 * F), jnp.float32)
    for e in range(E):
        eh = jax.nn.gelu(jnp.dot(x, We1_ref[e], preferred_element_type=jnp.float32) + be1_ref[e])
        eo = jnp.dot(eh, We2_ref[e], preferred_element_type=jnp.float32) + be2_ref[e]
        moe = moe + eo * w[:, e:e + 1]

    t1 = jnp.maximum(jnp.dot(moe, Wt1a_ref[...], preferred_element_type=jnp.float32) + bt1a_ref[...], 0.0)
    t1 = jnp.dot(t1, Wt1b_ref[...], preferred_element_type=jnp.float32) + bt1b_ref[...]
    t2 = jnp.maximum(jnp.dot(moe, Wt2a_ref[...], preferred_element_type=jnp.float32) + bt2a_ref[...], 0.0)
    t2 = jnp.dot(t2, Wt2b_ref[...], preferred_element_type=jnp.float32) + bt2b_ref[...]
    out_ref[:, 0:1] = t1
    out_ref[:, 1:2] = t2


def kernel(ts_x, bert_x, params):
    p = params
    r1 = lambda a: a.reshape(1, -1)
    Wxh = jnp.concatenate([p['Wx'], p['Wh']], axis=0)

    BB1 = 256
    hs = pl.pallas_call(
        _conv_lstm_body,
        grid=(B // BB1,),
        in_specs=[
            pl.BlockSpec((BB1, T, CIN), lambda i: (i, 0, 0)),
            _fullspec((3, CIN, F)), _fullspec((1, F)),
            _fullspec((3, F, F)), _fullspec((1, F)),
            _fullspec((2 * F, H4)), _fullspec((1, H4)),
        ],
        out_specs=pl.BlockSpec((BB1, T, F), lambda i: (i, 0, 0)),
        out_shape=jax.ShapeDtypeStruct((B, T, F), jnp.float32),
        scratch_shapes=[pltpu.VMEM((BB1, T, F), jnp.float32)],
    )(ts_x, p['Wc1'], r1(p['bc1']), p['Wc2'], r1(p['bc2']),
      Wxh, r1(p['bl']))

    BB1B = 128
    ts_feat = pl.pallas_call(
        _mha_ffn_body,
        grid=(B // BB1B,),
        in_specs=[
            pl.BlockSpec((BB1B, T, F), lambda i: (i, 0, 0)),
            _fullspec((F, F)), _fullspec((F, F)), _fullspec((F, F)), _fullspec((F, F)),
            _fullspec((1, F)), _fullspec((1, F)),
            _fullspec((F, DFF)), _fullspec((1, DFF)),
            _fullspec((DFF, F)), _fullspec((1, F)),
            _fullspec((1, F)), _fullspec((1, F)),
            _fullspec((F, F)), _fullspec((1, F)),
        ],
        out_specs=pl.BlockSpec((BB1B, F), lambda i: (i, 0)),
        out_shape=jax.ShapeDtypeStruct((B, F), jnp.float32),
    )(hs, p['Wq'], p['Wk'], p['Wv'], p['Wo'],
      r1(p['ln1g']), r1(p['ln1b']), p['Wf1'], r1(p['bf1']), p['Wf2'], r1(p['bf2']),
      r1(p['ln2g']), r1(p['ln2b']), p['Wts'], r1(p['bts']))

    BB2 = 256
    xcat, logits = pl.pallas_call(
        _gru_branch_body,
        grid=(B // BB2,),
        in_specs=[
            pl.BlockSpec((BB2, S, BF), lambda i: (i, 0, 0)),
            pl.BlockSpec((BB2, F), lambda i: (i, 0)),
            _fullspec((BF, G3)), _fullspec((1, G3)), _fullspec((F, G3)),
            _fullspec((F, F)), _fullspec((1, F)),
            _fullspec((2 * F, E)), _fullspec((1, E)),
        ],
        out_specs=[
            pl.BlockSpec((BB2, 2 * F), lambda i: (i, 0)),
            pl.BlockSpec((BB2, E), lambda i: (i, 0)),
        ],
        out_shape=[
            jax.ShapeDtypeStruct((B, 2 * F), jnp.float32),
            jax.ShapeDtypeStruct((B, E), jnp.float32),
        ],
        scratch_shapes=[pltpu.VMEM((BB2, S, G3), jnp.float32)],
    )(bert_x, ts_feat, p['Wgx'], r1(p['bgr']), p['Wgh'],
      p['Wp'], r1(p['bp']), p['Wg'], r1(p['bg']))

    w = _router_sc(logits)

    BB4 = 256
    out = pl.pallas_call(
        _moe_body,
        grid=(B // BB4,),
        in_specs=[
            pl.BlockSpec((BB4, 2 * F), lambda i: (i, 0)),
            pl.BlockSpec((BB4, E), lambda i: (i, 0)),
            _fullspec((E, 2 * F, DE)), _fullspec((E, DE)),
            _fullspec((E, DE, 2 * F)), _fullspec((E, 2 * F)),
            _fullspec((2 * F, F)), _fullspec((1, F)), _fullspec((F, 1)), _fullspec((1, 1)),
            _fullspec((2 * F, F)), _fullspec((1, F)), _fullspec((F, 1)), _fullspec((1, 1)),
        ],
        out_specs=pl.BlockSpec((BB4, 2), lambda i: (i, 0)),
        out_shape=jax.ShapeDtypeStruct((B, 2), jnp.float32),
    )(xcat, w, p['We1'], p['be1'], p['We2'], p['be2'],
      p['Wt1a'], r1(p['bt1a']), p['Wt1b'], r1(p['bt1b']),
      p['Wt2a'], r1(p['bt2a']), p['Wt2b'], r1(p['bt2b']))
    return out


# speedup vs baseline: 1.2183x; 1.2183x over previous
import jax
import jax.numpy as jnp
import numpy as np
from jax import lax
from jax.experimental import pallas as pl
from jax.experimental.pallas import tpu as pltpu
from jax.experimental.pallas import tpu_sc as plsc

F = 128
T = 64
CIN = 64
S = 20
BF = 768
B = 1024
E = 8
H4 = 512
G3 = 384
DFF = 256
DE = 512


def _fullspec(shape):
    nd = len(shape)
    return pl.BlockSpec(shape, lambda i, _n=nd: (0,) * _n)


def _lnorm(x, g, b):
    m = jnp.mean(x, axis=-1, keepdims=True)
    xc = x - m
    v = jnp.mean(xc * xc, axis=-1, keepdims=True)
    return xc / jnp.sqrt(v + 1e-5) * g + b


def _softmax_last(x):
    m = jnp.max(x, axis=-1, keepdims=True)
    e = jnp.exp(x - m)
    return e / jnp.sum(e, axis=-1, keepdims=True)


def _ts_branch_body(ts_ref, Wc1_ref, bc1_ref, Wc2_ref, bc2_ref,
                    Wx_ref, bl_ref, Wh_ref,
                    Wqkv_ref, Wo_ref,
                    ln1g_ref, ln1b_ref, Wf1_ref, bf1_ref, Wf2_ref, bf2_ref,
                    ln2g_ref, ln2b_ref, Wts_ref, bts_ref,
                    out_ref, xz_ref, hs_ref):
    BB = ts_ref.shape[1]
    x = ts_ref[...]

    xf = x.reshape(T * BB, CIN)
    z0 = jnp.dot(xf, Wc1_ref[0], preferred_element_type=jnp.float32).reshape(T, BB, F)
    z1 = jnp.dot(xf, Wc1_ref[1], preferred_element_type=jnp.float32).reshape(T, BB, F)
    z2 = jnp.dot(xf, Wc1_ref[2], preferred_element_type=jnp.float32).reshape(T, BB, F)
    y = z2
    y = y + jnp.concatenate([jnp.zeros((1, BB, F), jnp.float32), z1[: T - 1]], axis=0)
    y = y + jnp.concatenate([jnp.zeros((2, BB, F), jnp.float32), z0[: T - 2]], axis=0)
    h1 = jnp.maximum(y + bc1_ref[...], 0.0)

    hf = h1.reshape(T * BB, F)
    z0 = jnp.dot(hf, Wc2_ref[0], preferred_element_type=jnp.float32).reshape(T, BB, F)
    z1 = jnp.dot(hf, Wc2_ref[1], preferred_element_type=jnp.float32).reshape(T, BB, F)
    z2 = jnp.dot(hf, Wc2_ref[2], preferred_element_type=jnp.float32).reshape(T, BB, F)
    y = z2
    y = y + jnp.concatenate([jnp.zeros((2, BB, F), jnp.float32), z1[: T - 2]], axis=0)
    y = y + jnp.concatenate([jnp.zeros((4, BB, F), jnp.float32), z0[: T - 4]], axis=0)
    h2 = jnp.maximum(y + bc2_ref[...], 0.0)

    CH = 16
    for c in range(T // CH):
        blk = h2[c * CH:(c + 1) * CH].reshape(CH * BB, F)
        xz = jnp.dot(blk, Wx_ref[...], preferred_element_type=jnp.float32) + bl_ref[...]
        xz_ref[c * CH:(c + 1) * CH] = xz.reshape(CH, BB, H4)

    Wh = Wh_ref[...]

    def step(t, carry):
        h, cst = carry
        zt = xz_ref[t] + jnp.dot(h, Wh, preferred_element_type=jnp.float32)
        ig = jax.nn.sigmoid(zt[:, 0:F])
        fg = jax.nn.sigmoid(zt[:, F:2 * F])
        gg = jnp.tanh(zt[:, 2 * F:3 * F])
        og = jax.nn.sigmoid(zt[:, 3 * F:4 * F])
        cst = fg * cst + ig * gg
        h = og * jnp.tanh(cst)
        hs_ref[:, pl.ds(t, 1), :] = h[:, None, :]
        return (h, cst)

    zero = jnp.zeros((BB, F), jnp.float32)
    lax.fori_loop(0, T, step, (zero, zero))

    hs = hs_ref[...]
    hflat = hs.reshape(BB * T, F)

    qkv = jnp.dot(hflat, Wqkv_ref[...], preferred_element_type=jnp.float32).reshape(BB, T, 3 * F)
    hd = F // 4
    heads = []
    for hh in range(4):
        qh = qkv[:, :, hh * hd:(hh + 1) * hd]
        kh = qkv[:, :, F + hh * hd:F + (hh + 1) * hd]
        vh = qkv[:, :, 2 * F + hh * hd:2 * F + (hh + 1) * hd]
        s = lax.dot_general(qh, kh, dimension_numbers=(((2,), (2,)), ((0,), (0,))),
                            preferred_element_type=jnp.float32)
        a = _softmax_last(s)
        oh = lax.dot_general(a, vh, dimension_numbers=(((2,), (1,)), ((0,), (0,))),
                             preferred_element_type=jnp.float32)
        heads.append(oh)
    mh = jnp.concatenate(heads, axis=-1).reshape(BB * T, F)
    mo = jnp.dot(mh, Wo_ref[...], preferred_element_type=jnp.float32)

    h = _lnorm(hflat + mo, ln1g_ref[...], ln1b_ref[...])
    f1 = jnp.maximum(jnp.dot(h, Wf1_ref[...], preferred_element_type=jnp.float32) + bf1_ref[...], 0.0)
    f2 = jnp.dot(f1, Wf2_ref[...], preferred_element_type=jnp.float32) + bf2_ref[...]
    h = _lnorm(h + f2, ln2g_ref[...], ln2b_ref[...])

    pooled = jnp.mean(h.reshape(BB, T, F), axis=1)
    out_ref[...] = jnp.dot(pooled, Wts_ref[...], preferred_element_type=jnp.float32) + bts_ref[...]


def _gru_branch_body(bert_ref, tsf_ref, Wgx_ref, bgr_ref, Wgh_ref,
                     Wp_ref, bp_ref, Wg_ref, bg_ref,
                     x_ref, logit_ref, xl_ref):
    BB = bert_ref.shape[1]
    bx = bert_ref[...].reshape(S * BB, BF)
    xl = jnp.dot(bx, Wgx_ref[...], preferred_element_type=jnp.float32) + bgr_ref[...]
    xl_ref[...] = xl.reshape(S, BB, G3)

    Wgh = Wgh_ref[...]

    def step(t, h):
        xlt = xl_ref[t]
        hl = jnp.dot(h, Wgh, preferred_element_type=jnp.float32)
        r = jax.nn.sigmoid(xlt[:, 0:F] + hl[:, 0:F])
        zz = jax.nn.sigmoid(xlt[:, F:2 * F] + hl[:, F:2 * F])
        n = jnp.tanh(xlt[:, 2 * F:3 * F] + r * hl[:, 2 * F:3 * F])
        return (1.0 - zz) * n + zz * h

    h = lax.fori_loop(0, S, step, jnp.zeros((BB, F), jnp.float32))
    text_feat = jnp.dot(h, Wp_ref[...], preferred_element_type=jnp.float32) + bp_ref[...]

    x_ref[:, 0:F] = tsf_ref[...]
    x_ref[:, F:2 * F] = text_feat
    xcat = x_ref[...]
    logit_ref[...] = jnp.dot(xcat, Wg_ref[...], preferred_element_type=jnp.float32) + bg_ref[...]


def _router_sc_body(lt_hbm, w_hbm, lt_v, w_v):
    wid = lax.axis_index("s") * 2 + lax.axis_index("c")
    rows = B // 32
    base = wid * rows * E
    pltpu.sync_copy(lt_hbm.at[pl.ds(base, rows * E)], lt_v)
    for c in range(rows // 16):
        ls = [lt_v[pl.ds(e * rows + c * 16, 16)] for e in range(E)]
        m = ls[0]
        for e in range(1, E):
            m = jnp.maximum(m, ls[e])
        ex = [jnp.exp(l - m) for l in ls]
        ssum = ex[0]
        for e in range(1, E):
            ssum = ssum + ex[e]
        ps = [v / ssum for v in ex]
        v1 = ps[0]
        for e in range(1, E):
            v1 = jnp.maximum(v1, ps[e])
        i1 = jnp.full((16,), E, jnp.int32)
        for e in reversed(range(E)):
            i1 = jnp.where(ps[e] == v1, e, i1)
        p2 = [jnp.where(i1 == e, -1.0, ps[e]) for e in range(E)]
        v2 = p2[0]
        for e in range(1, E):
            v2 = jnp.maximum(v2, p2[e])
        i2 = jnp.full((16,), E, jnp.int32)
        for e in reversed(range(E)):
            i2 = jnp.where(p2[e] == v2, e, i2)
        denom = v1 + v2
        for e in range(E):
            w_v[pl.ds(e * rows + c * 16, 16)] = jnp.where(
                i1 == e, v1, jnp.where(i2 == e, v2, 0.0)) / denom
    pltpu.sync_copy(w_v, w_hbm.at[pl.ds(base, rows * E)])


def _router_sc(logits):
    rows = B // 32
    lt_blk = logits.T.reshape(E, 32, rows).transpose(1, 0, 2).reshape(-1)
    w_blk = pl.kernel(
        _router_sc_body,
        mesh=plsc.VectorSubcoreMesh(core_axis_name="c", subcore_axis_name="s"),
        out_type=jax.ShapeDtypeStruct((B * E,), jnp.float32),
        scratch_types=[
            pltpu.VMEM((rows * E,), jnp.float32),
            pltpu.VMEM((rows * E,), jnp.float32),
        ],
    )(lt_blk)
    return w_blk.reshape(32, E, rows).transpose(1, 0, 2).reshape(E, B).T


def _moe_body(x_ref, w_ref, We1_ref, be1_ref, We2_ref, be2_ref,
              Wt12a_ref, bt12a_ref, Wt12b_ref, bt12b_ref,
              out_ref):
    x = x_ref[...]
    w = w_ref[...]

    moe = jnp.zeros((x.shape[0], 2 * F), jnp.float32)
    for e in range(E):
        eh = jax.nn.gelu(jnp.dot(x, We1_ref[e], preferred_element_type=jnp.float32) + be1_ref[e])
        eo = jnp.dot(eh, We2_ref[e], preferred_element_type=jnp.float32) + be2_ref[e]
        moe = moe + eo * w[:, e:e + 1]

    tcat = jnp.maximum(jnp.dot(moe, Wt12a_ref[...], preferred_element_type=jnp.float32) + bt12a_ref[...], 0.0)
    out_ref[...] = jnp.dot(tcat, Wt12b_ref[...], preferred_element_type=jnp.float32) + bt12b_ref[...]


def kernel(ts_x, bert_x, params):
    p = params
    ts_tm = jnp.swapaxes(ts_x, 0, 1)
    bert_tm = jnp.swapaxes(bert_x, 0, 1)

    r1 = lambda a: a.reshape(1, -1)
    hd = F // 4
    Wqkv = jnp.concatenate(
        [p['Wq'] * np.float32(1.0 / np.sqrt(hd)), p['Wk'], p['Wv']], axis=1)

    BB1 = 128
    ts_feat = pl.pallas_call(
        _ts_branch_body,
        grid=(B // BB1,),
        in_specs=[
            pl.BlockSpec((T, BB1, CIN), lambda i: (0, i, 0)),
            _fullspec((3, CIN, F)), _fullspec((1, F)),
            _fullspec((3, F, F)), _fullspec((1, F)),
            _fullspec((F, H4)), _fullspec((1, H4)), _fullspec((F, H4)),
            _fullspec((F, 3 * F)), _fullspec((F, F)),
            _fullspec((1, F)), _fullspec((1, F)),
            _fullspec((F, DFF)), _fullspec((1, DFF)),
            _fullspec((DFF, F)), _fullspec((1, F)),
            _fullspec((1, F)), _fullspec((1, F)),
            _fullspec((F, F)), _fullspec((1, F)),
        ],
        out_specs=pl.BlockSpec((BB1, F), lambda i: (i, 0)),
        out_shape=jax.ShapeDtypeStruct((B, F), jnp.float32),
        scratch_shapes=[
            pltpu.VMEM((T, BB1, H4), jnp.float32),
            pltpu.VMEM((BB1, T, F), jnp.float32),
        ],
    )(ts_tm, p['Wc1'], r1(p['bc1']), p['Wc2'], r1(p['bc2']),
      p['Wx'], r1(p['bl']), p['Wh'],
      Wqkv, p['Wo'],
      r1(p['ln1g']), r1(p['ln1b']), p['Wf1'], r1(p['bf1']), p['Wf2'], r1(p['bf2']),
      r1(p['ln2g']), r1(p['ln2b']), p['Wts'], r1(p['bts']))

    BB2 = 128
    xcat, logits = pl.pallas_call(
        _gru_branch_body,
        grid=(B // BB2,),
        in_specs=[
            pl.BlockSpec((S, BB2, BF), lambda i: (0, i, 0)),
            pl.BlockSpec((BB2, F), lambda i: (i, 0)),
            _fullspec((BF, G3)), _fullspec((1, G3)), _fullspec((F, G3)),
            _fullspec((F, F)), _fullspec((1, F)),
            _fullspec((2 * F, E)), _fullspec((1, E)),
        ],
        out_specs=[
            pl.BlockSpec((BB2, 2 * F), lambda i: (i, 0)),
            pl.BlockSpec((BB2, E), lambda i: (i, 0)),
        ],
        out_shape=[
            jax.ShapeDtypeStruct((B, 2 * F), jnp.float32),
            jax.ShapeDtypeStruct((B, E), jnp.float32),
        ],
        scratch_shapes=[pltpu.VMEM((S, BB2, G3), jnp.float32)],
    )(bert_tm, ts_feat, p['Wgx'], r1(p['bgr']), p['Wgh'],
      p['Wp'], r1(p['bp']), p['Wg'], r1(p['bg']))

    w = _router_sc(logits)

    BB4 = 256
    out = pl.pallas_call(
        _moe_body,
        grid=(B // BB4,),
        in_specs=[
            pl.BlockSpec((BB4, 2 * F), lambda i: (i, 0)),
            pl.BlockSpec((BB4, E), lambda i: (i, 0)),
            _fullspec((E, 2 * F, DE)), _fullspec((E, DE)),
            _fullspec((E, DE, 2 * F)), _fullspec((E, 2 * F)),
            _fullspec((2 * F, 2 * F)), _fullspec((1, 2 * F)),
            _fullspec((2 * F, 2)), _fullspec((1, 2)),
        ],
        out_specs=pl.BlockSpec((BB4, 2), lambda i: (i, 0)),
        out_shape=jax.ShapeDtypeStruct((B, 2), jnp.float32),
    )(xcat, w, p['We1'], p['be1'], p['We2'], p['be2'],
      jnp.concatenate([p['Wt1a'], p['Wt2a']], axis=1),
      jnp.concatenate([p['bt1a'], p['bt2a']]).reshape(1, -1),
      jnp.concatenate([
          jnp.concatenate([p['Wt1b'], jnp.zeros((F, 1), jnp.float32)], axis=1),
          jnp.concatenate([jnp.zeros((F, 1), jnp.float32), p['Wt2b']], axis=1),
      ], axis=0),
      jnp.concatenate([p['bt1b'], p['bt2b']]).reshape(1, -1))
    return out


# speedup vs baseline: 1.2628x; 1.0365x over previous
import jax
import jax.numpy as jnp
import numpy as np
from jax import lax
from jax.experimental import pallas as pl
from jax.experimental.pallas import tpu as pltpu
from jax.experimental.pallas import tpu_sc as plsc

F = 128
T = 64
CIN = 64
S = 20
BF = 768
B = 1024
E = 8
H4 = 512
G3 = 384
DFF = 256
DE = 512


def _fullspec(shape):
    nd = len(shape)
    return pl.BlockSpec(shape, lambda i, _n=nd: (0,) * _n)


def _lnorm(x, g, b):
    m = jnp.mean(x, axis=-1, keepdims=True)
    xc = x - m
    v = jnp.mean(xc * xc, axis=-1, keepdims=True)
    return xc / jnp.sqrt(v + 1e-5) * g + b


def _softmax_last(x):
    m = jnp.max(x, axis=-1, keepdims=True)
    e = jnp.exp(x - m)
    return e / jnp.sum(e, axis=-1, keepdims=True)


def _conv_lstm_body(ts_ref, Wc1_ref, bc1_ref, Wc2_ref, bc2_ref,
                    Wxh_ref, bl_ref, hs_ref, h2_ref):
    BB = ts_ref.shape[1]
    x = ts_ref[...]

    xf = x.reshape(T * BB, CIN)
    y = jnp.dot(xf, Wc1_ref[2], preferred_element_type=jnp.float32).reshape(T, BB, F) + bc1_ref[...]
    z1 = jnp.dot(xf, Wc1_ref[1], preferred_element_type=jnp.float32).reshape(T, BB, F)
    y = y + jnp.concatenate([jnp.zeros((1, BB, F), jnp.float32), z1[: T - 1]], axis=0)
    z0 = jnp.dot(xf, Wc1_ref[0], preferred_element_type=jnp.float32).reshape(T, BB, F)
    y = y + jnp.concatenate([jnp.zeros((2, BB, F), jnp.float32), z0[: T - 2]], axis=0)
    h1 = jnp.maximum(y, 0.0)

    hf = h1.reshape(T * BB, F)
    y = jnp.dot(hf, Wc2_ref[2], preferred_element_type=jnp.float32).reshape(T, BB, F) + bc2_ref[...]
    z1 = jnp.dot(hf, Wc2_ref[1], preferred_element_type=jnp.float32).reshape(T, BB, F)
    y = y + jnp.concatenate([jnp.zeros((2, BB, F), jnp.float32), z1[: T - 2]], axis=0)
    z0 = jnp.dot(hf, Wc2_ref[0], preferred_element_type=jnp.float32).reshape(T, BB, F)
    y = y + jnp.concatenate([jnp.zeros((4, BB, F), jnp.float32), z0[: T - 4]], axis=0)
    h2_ref[...] = jnp.maximum(y, 0.0)

    Wxh = Wxh_ref[...]
    bl = bl_ref[...]

    def step(t, carry):
        h, cst = carry
        xt = h2_ref[t]
        zcat = jnp.concatenate([xt, h], axis=-1)
        zt = jnp.dot(zcat, Wxh, preferred_element_type=jnp.float32) + bl
        ig = jax.nn.sigmoid(zt[:, 0:F])
        fg = jax.nn.sigmoid(zt[:, F:2 * F])
        gg = jnp.tanh(zt[:, 2 * F:3 * F])
        og = jax.nn.sigmoid(zt[:, 3 * F:4 * F])
        cst = fg * cst + ig * gg
        h = og * jnp.tanh(cst)
        hs_ref[:, pl.ds(t, 1), :] = h[:, None, :]
        return (h, cst)

    zero = jnp.zeros((BB, F), jnp.float32)
    lax.fori_loop(0, T, step, (zero, zero))


def _mha_ffn_body(hs_ref, Wqkv_ref, Wo_ref,
                  ln1g_ref, ln1b_ref, Wf1_ref, bf1_ref, Wf2_ref, bf2_ref,
                  ln2g_ref, ln2b_ref, Wts_ref, bts_ref, out_ref):
    BB = hs_ref.shape[0]
    hs = hs_ref[...]
    hflat = hs.reshape(BB * T, F)

    qkv = jnp.dot(hflat, Wqkv_ref[...], preferred_element_type=jnp.float32).reshape(BB, T, 3 * F)
    hd = F // 4
    heads = []
    for hh in range(4):
        qh = qkv[:, :, hh * hd:(hh + 1) * hd]
        kh = qkv[:, :, F + hh * hd:F + (hh + 1) * hd]
        vh = qkv[:, :, 2 * F + hh * hd:2 * F + (hh + 1) * hd]
        s = lax.dot_general(qh, kh, dimension_numbers=(((2,), (2,)), ((0,), (0,))),
                            preferred_element_type=jnp.float32)
        a = _softmax_last(s)
        oh = lax.dot_general(a, vh, dimension_numbers=(((2,), (1,)), ((0,), (0,))),
                             preferred_element_type=jnp.float32)
        heads.append(oh)
    mh = jnp.concatenate(heads, axis=-1).reshape(BB * T, F)
    mo = jnp.dot(mh, Wo_ref[...], preferred_element_type=jnp.float32)

    h = _lnorm(hflat + mo, ln1g_ref[...], ln1b_ref[...])
    f1 = jnp.maximum(jnp.dot(h, Wf1_ref[...], preferred_element_type=jnp.float32) + bf1_ref[...], 0.0)
    f2 = jnp.dot(f1, Wf2_ref[...], preferred_element_type=jnp.float32) + bf2_ref[...]
    h = _lnorm(h + f2, ln2g_ref[...], ln2b_ref[...])

    pooled = jnp.mean(h.reshape(BB, T, F), axis=1)
    out_ref[...] = jnp.dot(pooled, Wts_ref[...], preferred_element_type=jnp.float32) + bts_ref[...]


def _gru_branch_body(bert_ref, tsf_ref, Wgx_ref, bgr_ref, Wgh_ref,
                     Wp_ref, bp_ref, Wg_ref, bg_ref,
                     x_ref, logit_ref, xl_ref):
    BB = bert_ref.shape[1]
    bx = bert_ref[...].reshape(S * BB, BF)
    xl = jnp.dot(bx, Wgx_ref[...], preferred_element_type=jnp.float32) + bgr_ref[...]
    xl_ref[...] = xl.reshape(S, BB, G3)

    Wgh = Wgh_ref[...]

    def step(t, h):
        xlt = xl_ref[t]
        hl = jnp.dot(h, Wgh, preferred_element_type=jnp.float32)
        r = jax.nn.sigmoid(xlt[:, 0:F] + hl[:, 0:F])
        zz = jax.nn.sigmoid(xlt[:, F:2 * F] + hl[:, F:2 * F])
        n = jnp.tanh(xlt[:, 2 * F:3 * F] + r * hl[:, 2 * F:3 * F])
        return (1.0 - zz) * n + zz * h

    h = lax.fori_loop(0, S, step, jnp.zeros((BB, F), jnp.float32))
    text_feat = jnp.dot(h, Wp_ref[...], preferred_element_type=jnp.float32) + bp_ref[...]

    x_ref[:, 0:F] = tsf_ref[...]
    x_ref[:, F:2 * F] = text_feat
    xcat = x_ref[...]
    logit_ref[...] = jnp.dot(xcat, Wg_ref[...], preferred_element_type=jnp.float32) + bg_ref[...]


def _router_sc_body(lt_hbm, w_hbm, lt_v, w_v):
    wid = lax.axis_index("s") * 2 + lax.axis_index("c")
    rows = B // 32
    base = wid * rows * E
    pltpu.sync_copy(lt_hbm.at[pl.ds(base, rows * E)], lt_v)
    for c in range(rows // 16):
        ls = [lt_v[pl.ds(e * rows + c * 16, 16)] for e in range(E)]
        m = ls[0]
        for e in range(1, E):
            m = jnp.maximum(m, ls[e])
        ex = [jnp.exp(l - m) for l in ls]
        ssum = ex[0]
        for e in range(1, E):
            ssum = ssum + ex[e]
        ps = [v / ssum for v in ex]
        v1 = ps[0]
        for e in range(1, E):
            v1 = jnp.maximum(v1, ps[e])
        i1 = jnp.full((16,), E, jnp.int32)
        for e in reversed(range(E)):
            i1 = jnp.where(ps[e] == v1, e, i1)
        p2 = [jnp.where(i1 == e, -1.0, ps[e]) for e in range(E)]
        v2 = p2[0]
        for e in range(1, E):
            v2 = jnp.maximum(v2, p2[e])
        i2 = jnp.full((16,), E, jnp.int32)
        for e in reversed(range(E)):
            i2 = jnp.where(p2[e] == v2, e, i2)
        denom = v1 + v2
        for e in range(E):
            w_v[pl.ds(e * rows + c * 16, 16)] = jnp.where(
                i1 == e, v1, jnp.where(i2 == e, v2, 0.0)) / denom
    pltpu.sync_copy(w_v, w_hbm.at[pl.ds(base, rows * E)])


def _router_sc(logits):
    rows = B // 32
    lt_blk = logits.T.reshape(E, 32, rows).transpose(1, 0, 2).reshape(-1)
    w_blk = pl.kernel(
        _router_sc_body,
        mesh=plsc.VectorSubcoreMesh(core_axis_name="c", subcore_axis_name="s"),
        out_type=jax.ShapeDtypeStruct((B * E,), jnp.float32),
        scratch_types=[
            pltpu.VMEM((rows * E,), jnp.float32),
            pltpu.VMEM((rows * E,), jnp.float32),
        ],
    )(lt_blk)
    return w_blk.reshape(32, E, rows).transpose(1, 0, 2).reshape(E, B).T


def _moe_body(x_ref, w_ref, We1_ref, be1_ref, We2_ref, be2_ref,
              Wt12a_ref, bt12a_ref, Wt12b_ref, bt12b_ref,
              out_ref):
    x = x_ref[...]
    w = w_ref[...]

    xb = x.astype(jnp.bfloat16)
    moe = jnp.zeros((x.shape[0], 2 * F), jnp.float32)
    for e in range(E):
        eh = jax.nn.gelu(jnp.dot(xb, We1_ref[e].astype(jnp.bfloat16),
                                 preferred_element_type=jnp.float32) + be1_ref[e])
        eo = jnp.dot(eh.astype(jnp.bfloat16), We2_ref[e].astype(jnp.bfloat16),
                     preferred_element_type=jnp.float32) + be2_ref[e]
        moe = moe + eo * w[:, e:e + 1]

    tcat = jnp.maximum(jnp.dot(moe, Wt12a_ref[...], preferred_element_type=jnp.float32) + bt12a_ref[...], 0.0)
    out_ref[...] = jnp.dot(tcat, Wt12b_ref[...], preferred_element_type=jnp.float32) + bt12b_ref[...]


def kernel(ts_x, bert_x, params):
    p = params
    ts_tm = jnp.swapaxes(ts_x, 0, 1)
    bert_tm = jnp.swapaxes(bert_x, 0, 1)

    r1 = lambda a: a.reshape(1, -1)
    hd = F // 4
    Wqkv = jnp.concatenate(
        [p['Wq'] * np.float32(1.0 / np.sqrt(hd)), p['Wk'], p['Wv']], axis=1)

    Wxh = jnp.concatenate([p['Wx'], p['Wh']], axis=0)

    BB1A = 256
    hs = pl.pallas_call(
        _conv_lstm_body,
        grid=(B // BB1A,),
        in_specs=[
            pl.BlockSpec((T, BB1A, CIN), lambda i: (0, i, 0)),
            _fullspec((3, CIN, F)), _fullspec((1, F)),
            _fullspec((3, F, F)), _fullspec((1, F)),
            _fullspec((2 * F, H4)), _fullspec((1, H4)),
        ],
        out_specs=pl.BlockSpec((BB1A, T, F), lambda i: (i, 0, 0)),
        out_shape=jax.ShapeDtypeStruct((B, T, F), jnp.float32),
        scratch_shapes=[pltpu.VMEM((T, BB1A, F), jnp.float32)],
    )(ts_tm, p['Wc1'], r1(p['bc1']), p['Wc2'], r1(p['bc2']),
      Wxh, r1(p['bl']))

    BB1 = 128
    ts_feat = pl.pallas_call(
        _mha_ffn_body,
        grid=(B // BB1,),
        in_specs=[
            pl.BlockSpec((BB1, T, F), lambda i: (i, 0, 0)),
            _fullspec((F, 3 * F)), _fullspec((F, F)),
            _fullspec((1, F)), _fullspec((1, F)),
            _fullspec((F, DFF)), _fullspec((1, DFF)),
            _fullspec((DFF, F)), _fullspec((1, F)),
            _fullspec((1, F)), _fullspec((1, F)),
            _fullspec((F, F)), _fullspec((1, F)),
        ],
        out_specs=pl.BlockSpec((BB1, F), lambda i: (i, 0)),
        out_shape=jax.ShapeDtypeStruct((B, F), jnp.float32),
    )(hs, Wqkv, p['Wo'],
      r1(p['ln1g']), r1(p['ln1b']), p['Wf1'], r1(p['bf1']), p['Wf2'], r1(p['bf2']),
      r1(p['ln2g']), r1(p['ln2b']), p['Wts'], r1(p['bts']))

    BB2 = 128
    xcat, logits = pl.pallas_call(
        _gru_branch_body,
        grid=(B // BB2,),
        in_specs=[
            pl.BlockSpec((S, BB2, BF), lambda i: (0, i, 0)),
            pl.BlockSpec((BB2, F), lambda i: (i, 0)),
            _fullspec((BF, G3)), _fullspec((1, G3)), _fullspec((F, G3)),
            _fullspec((F, F)), _fullspec((1, F)),
            _fullspec((2 * F, E)), _fullspec((1, E)),
        ],
        out_specs=[
            pl.BlockSpec((BB2, 2 * F), lambda i: (i, 0)),
            pl.BlockSpec((BB2, E), lambda i: (i, 0)),
        ],
        out_shape=[
            jax.ShapeDtypeStruct((B, 2 * F), jnp.float32),
            jax.ShapeDtypeStruct((B, E), jnp.float32),
        ],
        scratch_shapes=[pltpu.VMEM((S, BB2, G3), jnp.float32)],
    )(bert_tm, ts_feat, p['Wgx'], r1(p['bgr']), p['Wgh'],
      p['Wp'], r1(p['bp']), p['Wg'], r1(p['bg']))

    w = _router_sc(logits)

    BB4 = 256
    out = pl.pallas_call(
        _moe_body,
        grid=(B // BB4,),
        in_specs=[
            pl.BlockSpec((BB4, 2 * F), lambda i: (i, 0)),
            pl.BlockSpec((BB4, E), lambda i: (i, 0)),
            _fullspec((E, 2 * F, DE)), _fullspec((E, DE)),
            _fullspec((E, DE, 2 * F)), _fullspec((E, 2 * F)),
            _fullspec((2 * F, 2 * F)), _fullspec((1, 2 * F)),
            _fullspec((2 * F, 2)), _fullspec((1, 2)),
        ],
        out_specs=pl.BlockSpec((BB4, 2), lambda i: (i, 0)),
        out_shape=jax.ShapeDtypeStruct((B, 2), jnp.float32),
    )(xcat, w, p['We1'], p['be1'], p['We2'], p['be2'],
      jnp.concatenate([p['Wt1a'], p['Wt2a']], axis=1),
      jnp.concatenate([p['bt1a'], p['bt2a']]).reshape(1, -1),
      jnp.concatenate([
          jnp.concatenate([p['Wt1b'], jnp.zeros((F, 1), jnp.float32)], axis=1),
          jnp.concatenate([jnp.zeros((F, 1), jnp.float32), p['Wt2b']], axis=1),
      ], axis=0),
      jnp.concatenate([p['bt1b'], p['bt2b']]).reshape(1, -1))
    return out


# speedup vs baseline: 1.2966x; 1.0267x over previous
import jax
import jax.numpy as jnp
import numpy as np
from jax import lax
from jax.experimental import pallas as pl
from jax.experimental.pallas import tpu as pltpu
from jax.experimental.pallas import tpu_sc as plsc

F = 128
T = 64
CIN = 64
S = 20
BF = 768
B = 1024
E = 8
H4 = 512
G3 = 384
DFF = 256
DE = 512


def _fullspec(shape):
    nd = len(shape)
    return pl.BlockSpec(shape, lambda i, _n=nd: (0,) * _n)


def _lnorm(x, g, b):
    m = jnp.mean(x, axis=-1, keepdims=True)
    xc = x - m
    v = jnp.mean(xc * xc, axis=-1, keepdims=True)
    return xc / jnp.sqrt(v + 1e-5) * g + b


def _softmax_last(x):
    m = jnp.max(x, axis=-1, keepdims=True)
    e = jnp.exp(x - m)
    return e / jnp.sum(e, axis=-1, keepdims=True)


def _conv_lstm_body(ts_ref, Wc1_ref, bc1_ref, Wc2_ref, bc2_ref,
                    Wxh_ref, bl_ref, hs_ref, h2_ref):
    BB = ts_ref.shape[1]
    x = ts_ref[...]

    xf = x.reshape(T * BB, CIN)
    y = jnp.dot(xf, Wc1_ref[2], preferred_element_type=jnp.float32).reshape(T, BB, F) + bc1_ref[...]
    z1 = jnp.dot(xf, Wc1_ref[1], preferred_element_type=jnp.float32).reshape(T, BB, F)
    y = y + jnp.concatenate([jnp.zeros((1, BB, F), jnp.float32), z1[: T - 1]], axis=0)
    z0 = jnp.dot(xf, Wc1_ref[0], preferred_element_type=jnp.float32).reshape(T, BB, F)
    y = y + jnp.concatenate([jnp.zeros((2, BB, F), jnp.float32), z0[: T - 2]], axis=0)
    h1 = jnp.maximum(y, 0.0)

    hf = h1.reshape(T * BB, F)
    y = jnp.dot(hf, Wc2_ref[2], preferred_element_type=jnp.float32).reshape(T, BB, F) + bc2_ref[...]
    z1 = jnp.dot(hf, Wc2_ref[1], preferred_element_type=jnp.float32).reshape(T, BB, F)
    y = y + jnp.concatenate([jnp.zeros((2, BB, F), jnp.float32), z1[: T - 2]], axis=0)
    z0 = jnp.dot(hf, Wc2_ref[0], preferred_element_type=jnp.float32).reshape(T, BB, F)
    y = y + jnp.concatenate([jnp.zeros((4, BB, F), jnp.float32), z0[: T - 4]], axis=0)
    h2_ref[...] = jnp.maximum(y, 0.0)

    Wxh = Wxh_ref[...]
    bl = bl_ref[...]

    def step(t, carry):
        h, cst = carry
        xt = h2_ref[t]
        zcat = jnp.concatenate([xt, h], axis=-1)
        zt = jnp.dot(zcat, Wxh, preferred_element_type=jnp.float32) + bl
        ig = jax.nn.sigmoid(zt[:, 0:F])
        fg = jax.nn.sigmoid(zt[:, F:2 * F])
        gg = jnp.tanh(zt[:, 2 * F:3 * F])
        og = jax.nn.sigmoid(zt[:, 3 * F:4 * F])
        cst = fg * cst + ig * gg
        h = og * jnp.tanh(cst)
        hs_ref[:, pl.ds(t, 1), :] = h[:, None, :]
        return (h, cst)

    zero = jnp.zeros((BB, F), jnp.float32)
    lax.fori_loop(0, T, step, (zero, zero))


def _mha_ffn_body(hs_ref, Wqkv_ref, Wo_ref,
                  ln1g_ref, ln1b_ref, Wf1_ref, bf1_ref, Wf2_ref, bf2_ref,
                  ln2g_ref, ln2b_ref, Wts_ref, bts_ref, out_ref):
    BB = hs_ref.shape[0]
    hs = hs_ref[...]
    hflat = hs.reshape(BB * T, F)

    qkv = jnp.dot(hflat, Wqkv_ref[...], preferred_element_type=jnp.float32).reshape(BB, T, 3 * F)
    hd = F // 4
    heads = []
    for hh in range(4):
        qh = qkv[:, :, hh * hd:(hh + 1) * hd]
        kh = qkv[:, :, F + hh * hd:F + (hh + 1) * hd]
        vh = qkv[:, :, 2 * F + hh * hd:2 * F + (hh + 1) * hd]
        s = lax.dot_general(qh, kh, dimension_numbers=(((2,), (2,)), ((0,), (0,))),
                            preferred_element_type=jnp.float32)
        a = _softmax_last(s)
        oh = lax.dot_general(a, vh, dimension_numbers=(((2,), (1,)), ((0,), (0,))),
                             preferred_element_type=jnp.float32)
        heads.append(oh)
    mh = jnp.concatenate(heads, axis=-1).reshape(BB * T, F)
    mo = jnp.dot(mh, Wo_ref[...], preferred_element_type=jnp.float32)

    h = _lnorm(hflat + mo, ln1g_ref[...], ln1b_ref[...])
    f1 = jnp.maximum(jnp.dot(h, Wf1_ref[...], preferred_element_type=jnp.float32) + bf1_ref[...], 0.0)
    f2 = jnp.dot(f1, Wf2_ref[...], preferred_element_type=jnp.float32) + bf2_ref[...]
    h = _lnorm(h + f2, ln2g_ref[...], ln2b_ref[...])

    pooled = jnp.mean(h.reshape(BB, T, F), axis=1)
    out_ref[...] = jnp.dot(pooled, Wts_ref[...], preferred_element_type=jnp.float32) + bts_ref[...]


def _gru_branch_body(bert_ref, tsf_ref, Wgx_ref, bgr_ref, Wgh_ref,
                     Wp_ref, bp_ref, Wg_ref, bg_ref,
                     x_ref, logit_ref, xl_ref):
    BB = bert_ref.shape[1]
    bx = bert_ref[...].reshape(S * BB, BF)
    xl = jnp.dot(bx, Wgx_ref[...], preferred_element_type=jnp.float32) + bgr_ref[...]
    xl_ref[...] = xl.reshape(S, BB, G3)

    Wgh = Wgh_ref[...]

    def step(t, h):
        xlt = xl_ref[t]
        hl = jnp.dot(h, Wgh, preferred_element_type=jnp.float32)
        r = jax.nn.sigmoid(xlt[:, 0:F] + hl[:, 0:F])
        zz = jax.nn.sigmoid(xlt[:, F:2 * F] + hl[:, F:2 * F])
        n = jnp.tanh(xlt[:, 2 * F:3 * F] + r * hl[:, 2 * F:3 * F])
        return (1.0 - zz) * n + zz * h

    h = lax.fori_loop(0, S, step, jnp.zeros((BB, F), jnp.float32))
    text_feat = jnp.dot(h, Wp_ref[...], preferred_element_type=jnp.float32) + bp_ref[...]

    x_ref[:, 0:F] = tsf_ref[...]
    x_ref[:, F:2 * F] = text_feat
    xcat = x_ref[...]
    logit_ref[...] = jnp.dot(xcat, Wg_ref[...], preferred_element_type=jnp.float32) + bg_ref[...]


def _router_sc_body(lt_hbm, w_hbm, lt_v, w_v):
    wid = lax.axis_index("s") * 2 + lax.axis_index("c")
    rows = B // 32
    base = wid * rows * E
    pltpu.sync_copy(lt_hbm.at[pl.ds(base, rows * E)], lt_v)
    for c in range(rows // 16):
        ls = [lt_v[pl.ds(e * rows + c * 16, 16)] for e in range(E)]
        m = ls[0]
        for e in range(1, E):
            m = jnp.maximum(m, ls[e])
        ex = [jnp.exp(l - m) for l in ls]
        ssum = ex[0]
        for e in range(1, E):
            ssum = ssum + ex[e]
        ps = [v / ssum for v in ex]
        v1 = ps[0]
        for e in range(1, E):
            v1 = jnp.maximum(v1, ps[e])
        i1 = jnp.full((16,), E, jnp.int32)
        for e in reversed(range(E)):
            i1 = jnp.where(ps[e] == v1, e, i1)
        p2 = [jnp.where(i1 == e, -1.0, ps[e]) for e in range(E)]
        v2 = p2[0]
        for e in range(1, E):
            v2 = jnp.maximum(v2, p2[e])
        i2 = jnp.full((16,), E, jnp.int32)
        for e in reversed(range(E)):
            i2 = jnp.where(p2[e] == v2, e, i2)
        denom = v1 + v2
        for e in range(E):
            w_v[pl.ds(e * rows + c * 16, 16)] = jnp.where(
                i1 == e, v1, jnp.where(i2 == e, v2, 0.0)) / denom
    pltpu.sync_copy(w_v, w_hbm.at[pl.ds(base, rows * E)])


def _router_sc(logits):
    rows = B // 32
    lt_blk = logits.T.reshape(E, 32, rows).transpose(1, 0, 2).reshape(-1)
    w_blk = pl.kernel(
        _router_sc_body,
        mesh=plsc.VectorSubcoreMesh(core_axis_name="c", subcore_axis_name="s"),
        out_type=jax.ShapeDtypeStruct((B * E,), jnp.float32),
        scratch_types=[
            pltpu.VMEM((rows * E,), jnp.float32),
            pltpu.VMEM((rows * E,), jnp.float32),
        ],
    )(lt_blk)
    return w_blk.reshape(32, E, rows).transpose(1, 0, 2).reshape(E, B).T


def _moe_body(x_ref, w_ref, We1_ref, be1_ref, We2_ref, be2_ref,
              Wt12a_ref, bt12a_ref, Wt12b_ref, bt12b_ref,
              out_ref):
    x = x_ref[...]
    w = w_ref[...]

    xb = x.astype(jnp.bfloat16)
    moe = jnp.zeros((x.shape[0], 2 * F), jnp.float32)
    for e in range(E):
        eh = jax.nn.gelu(jnp.dot(xb, We1_ref[e].astype(jnp.bfloat16),
                                 preferred_element_type=jnp.float32) + be1_ref[e])
        eo = jnp.dot(eh.astype(jnp.bfloat16), We2_ref[e].astype(jnp.bfloat16),
                     preferred_element_type=jnp.float32) + be2_ref[e]
        moe = moe + eo * w[:, e:e + 1]

    tcat = jnp.maximum(jnp.dot(moe, Wt12a_ref[...], preferred_element_type=jnp.float32) + bt12a_ref[...], 0.0)
    out_ref[...] = jnp.dot(tcat, Wt12b_ref[...], preferred_element_type=jnp.float32) + bt12b_ref[...]


def kernel(ts_x, bert_x, params):
    p = params
    ts_tm = jnp.swapaxes(ts_x, 0, 1)
    bert_tm = jnp.swapaxes(bert_x, 0, 1)

    r1 = lambda a: a.reshape(1, -1)
    hd = F // 4
    Wqkv = jnp.concatenate(
        [p['Wq'] * np.float32(1.0 / np.sqrt(hd)), p['Wk'], p['Wv']], axis=1)

    Wxh = jnp.concatenate([p['Wx'], p['Wh']], axis=0)

    BB1A = 256
    hs = pl.pallas_call(
        _conv_lstm_body,
        grid=(B // BB1A,),
        in_specs=[
            pl.BlockSpec((T, BB1A, CIN), lambda i: (0, i, 0)),
            _fullspec((3, CIN, F)), _fullspec((1, F)),
            _fullspec((3, F, F)), _fullspec((1, F)),
            _fullspec((2 * F, H4)), _fullspec((1, H4)),
        ],
        out_specs=pl.BlockSpec((BB1A, T, F), lambda i: (i, 0, 0)),
        out_shape=jax.ShapeDtypeStruct((B, T, F), jnp.float32),
        scratch_shapes=[pltpu.VMEM((T, BB1A, F), jnp.float32)],
    )(ts_tm, p['Wc1'], r1(p['bc1']), p['Wc2'], r1(p['bc2']),
      Wxh, r1(p['bl']))

    BB1 = 128
    ts_feat = pl.pallas_call(
        _mha_ffn_body,
        grid=(B // BB1,),
        in_specs=[
            pl.BlockSpec((BB1, T, F), lambda i: (i, 0, 0)),
            _fullspec((F, 3 * F)), _fullspec((F, F)),
            _fullspec((1, F)), _fullspec((1, F)),
            _fullspec((F, DFF)), _fullspec((1, DFF)),
            _fullspec((DFF, F)), _fullspec((1, F)),
            _fullspec((1, F)), _fullspec((1, F)),
            _fullspec((F, F)), _fullspec((1, F)),
        ],
        out_specs=pl.BlockSpec((BB1, F), lambda i: (i, 0)),
        out_shape=jax.ShapeDtypeStruct((B, F), jnp.float32),
    )(hs, Wqkv, p['Wo'],
      r1(p['ln1g']), r1(p['ln1b']), p['Wf1'], r1(p['bf1']), p['Wf2'], r1(p['bf2']),
      r1(p['ln2g']), r1(p['ln2b']), p['Wts'], r1(p['bts']))

    BB2 = 256
    xcat, logits = pl.pallas_call(
        _gru_branch_body,
        grid=(B // BB2,),
        in_specs=[
            pl.BlockSpec((S, BB2, BF), lambda i: (0, i, 0)),
            pl.BlockSpec((BB2, F), lambda i: (i, 0)),
            _fullspec((BF, G3)), _fullspec((1, G3)), _fullspec((F, G3)),
            _fullspec((F, F)), _fullspec((1, F)),
            _fullspec((2 * F, E)), _fullspec((1, E)),
        ],
        out_specs=[
            pl.BlockSpec((BB2, 2 * F), lambda i: (i, 0)),
            pl.BlockSpec((BB2, E), lambda i: (i, 0)),
        ],
        out_shape=[
            jax.ShapeDtypeStruct((B, 2 * F), jnp.float32),
            jax.ShapeDtypeStruct((B, E), jnp.float32),
        ],
        scratch_shapes=[pltpu.VMEM((S, BB2, G3), jnp.float32)],
    )(bert_tm, ts_feat, p['Wgx'], r1(p['bgr']), p['Wgh'],
      p['Wp'], r1(p['bp']), p['Wg'], r1(p['bg']))

    w = _router_sc(logits)

    BB4 = 256
    out = pl.pallas_call(
        _moe_body,
        grid=(B // BB4,),
        in_specs=[
            pl.BlockSpec((BB4, 2 * F), lambda i: (i, 0)),
            pl.BlockSpec((BB4, E), lambda i: (i, 0)),
            _fullspec((E, 2 * F, DE)), _fullspec((E, DE)),
            _fullspec((E, DE, 2 * F)), _fullspec((E, 2 * F)),
            _fullspec((2 * F, 2 * F)), _fullspec((1, 2 * F)),
            _fullspec((2 * F, 2)), _fullspec((1, 2)),
        ],
        out_specs=pl.BlockSpec((BB4, 2), lambda i: (i, 0)),
        out_shape=jax.ShapeDtypeStruct((B, 2), jnp.float32),
    )(xcat, w, p['We1'], p['be1'], p['We2'], p['be2'],
      jnp.concatenate([p['Wt1a'], p['Wt2a']], axis=1),
      jnp.concatenate([p['bt1a'], p['bt2a']]).reshape(1, -1),
      jnp.concatenate([
          jnp.concatenate([p['Wt1b'], jnp.zeros((F, 1), jnp.float32)], axis=1),
          jnp.concatenate([jnp.zeros((F, 1), jnp.float32), p['Wt2b']], axis=1),
      ], axis=0),
      jnp.concatenate([p['bt1b'], p['bt2b']]).reshape(1, -1))
    return out


# speedup vs baseline: 1.2977x; 1.0009x over previous
import jax
import jax.numpy as jnp
import numpy as np
from jax import lax
from jax.experimental import pallas as pl
from jax.experimental.pallas import tpu as pltpu
from jax.experimental.pallas import tpu_sc as plsc

F = 128
T = 64
CIN = 64
S = 20
BF = 768
B = 1024
E = 8
H4 = 512
G3 = 384
DFF = 256
DE = 512


def _fullspec(shape):
    nd = len(shape)
    return pl.BlockSpec(shape, lambda i, _n=nd: (0,) * _n)


def _lnorm(x, g, b):
    m = jnp.mean(x, axis=-1, keepdims=True)
    xc = x - m
    v = jnp.mean(xc * xc, axis=-1, keepdims=True)
    return xc / jnp.sqrt(v + 1e-5) * g + b


def _softmax_last(x):
    m = jnp.max(x, axis=-1, keepdims=True)
    e = jnp.exp(x - m)
    return e / jnp.sum(e, axis=-1, keepdims=True)


def _conv_lstm_body(ts_ref, Wc1_ref, bc1_ref, Wc2_ref, bc2_ref,
                    Wxh_ref, bl_ref, hs_ref, h2_ref):
    BB = ts_ref.shape[1]
    x = ts_ref[...]

    xf = x.reshape(T * BB, CIN)
    y = jnp.dot(xf, Wc1_ref[2], preferred_element_type=jnp.float32).reshape(T, BB, F) + bc1_ref[...]
    z1 = jnp.dot(xf, Wc1_ref[1], preferred_element_type=jnp.float32).reshape(T, BB, F)
    y = y + jnp.concatenate([jnp.zeros((1, BB, F), jnp.float32), z1[: T - 1]], axis=0)
    z0 = jnp.dot(xf, Wc1_ref[0], preferred_element_type=jnp.float32).reshape(T, BB, F)
    y = y + jnp.concatenate([jnp.zeros((2, BB, F), jnp.float32), z0[: T - 2]], axis=0)
    h1 = jnp.maximum(y, 0.0)

    hf = h1.reshape(T * BB, F)
    y = jnp.dot(hf, Wc2_ref[2], preferred_element_type=jnp.float32).reshape(T, BB, F) + bc2_ref[...]
    z1 = jnp.dot(hf, Wc2_ref[1], preferred_element_type=jnp.float32).reshape(T, BB, F)
    y = y + jnp.concatenate([jnp.zeros((2, BB, F), jnp.float32), z1[: T - 2]], axis=0)
    z0 = jnp.dot(hf, Wc2_ref[0], preferred_element_type=jnp.float32).reshape(T, BB, F)
    y = y + jnp.concatenate([jnp.zeros((4, BB, F), jnp.float32), z0[: T - 4]], axis=0)
    h2_ref[...] = jnp.maximum(y, 0.0)

    Wxh = Wxh_ref[...]
    bl = bl_ref[...]

    def step(t, carry):
        h, cst = carry
        xt = h2_ref[t]
        zcat = jnp.concatenate([xt, h], axis=-1)
        zt = jnp.dot(zcat, Wxh, preferred_element_type=jnp.float32) + bl
        ig = jax.nn.sigmoid(zt[:, 0:F])
        fg = jax.nn.sigmoid(zt[:, F:2 * F])
        gg = jnp.tanh(zt[:, 2 * F:3 * F])
        og = jax.nn.sigmoid(zt[:, 3 * F:4 * F])
        cst = fg * cst + ig * gg
        h = og * jnp.tanh(cst)
        hs_ref[:, pl.ds(t, 1), :] = h[:, None, :]
        return (h, cst)

    zero = jnp.zeros((BB, F), jnp.float32)
    lax.fori_loop(0, T, step, (zero, zero))


def _mha_ffn_body(hs_ref, Wqkv_ref, Wo_ref,
                  ln1g_ref, ln1b_ref, Wf1_ref, bf1_ref, Wf2_ref, bf2_ref,
                  ln2g_ref, ln2b_ref, Wts_ref, bts_ref, out_ref):
    BB = hs_ref.shape[0]
    hs = hs_ref[...]
    hflat = hs.reshape(BB * T, F)

    qkv = jnp.dot(hflat, Wqkv_ref[...], preferred_element_type=jnp.float32).reshape(BB, T, 3 * F)
    hd = F // 4
    heads = []
    for hh in range(4):
        qh = qkv[:, :, hh * hd:(hh + 1) * hd]
        kh = qkv[:, :, F + hh * hd:F + (hh + 1) * hd]
        vh = qkv[:, :, 2 * F + hh * hd:2 * F + (hh + 1) * hd]
        s = lax.dot_general(qh, kh, dimension_numbers=(((2,), (2,)), ((0,), (0,))),
                            preferred_element_type=jnp.float32)
        a = _softmax_last(s)
        oh = lax.dot_general(a, vh, dimension_numbers=(((2,), (1,)), ((0,), (0,))),
                             preferred_element_type=jnp.float32)
        heads.append(oh)
    mh = jnp.concatenate(heads, axis=-1).reshape(BB * T, F)
    mo = jnp.dot(mh, Wo_ref[...], preferred_element_type=jnp.float32)

    h = _lnorm(hflat + mo, ln1g_ref[...], ln1b_ref[...])
    f1 = jnp.maximum(jnp.dot(h, Wf1_ref[...], preferred_element_type=jnp.float32) + bf1_ref[...], 0.0)
    f2 = jnp.dot(f1, Wf2_ref[...], preferred_element_type=jnp.float32) + bf2_ref[...]
    h = _lnorm(h + f2, ln2g_ref[...], ln2b_ref[...])

    pooled = jnp.mean(h.reshape(BB, T, F), axis=1)
    out_ref[...] = jnp.dot(pooled, Wts_ref[...], preferred_element_type=jnp.float32) + bts_ref[...]


def _gru_branch_body(bert_ref, tsf_ref, Wgx_ref, bgr_ref, Wgh_ref,
                     Wp_ref, bp_ref, Wg_ref, bg_ref,
                     x_ref, logit_ref, xl_ref):
    BB = bert_ref.shape[1]
    bx = bert_ref[...].reshape(S * BB, BF)
    xl = jnp.dot(bx, Wgx_ref[...], preferred_element_type=jnp.float32) + bgr_ref[...]
    xl_ref[...] = xl.reshape(S, BB, G3)

    Wgh = Wgh_ref[...]

    def step(t, h):
        xlt = xl_ref[t]
        hl = jnp.dot(h, Wgh, preferred_element_type=jnp.float32)
        r = jax.nn.sigmoid(xlt[:, 0:F] + hl[:, 0:F])
        zz = jax.nn.sigmoid(xlt[:, F:2 * F] + hl[:, F:2 * F])
        n = jnp.tanh(xlt[:, 2 * F:3 * F] + r * hl[:, 2 * F:3 * F])
        return (1.0 - zz) * n + zz * h

    h = lax.fori_loop(0, S, step, jnp.zeros((BB, F), jnp.float32))
    text_feat = jnp.dot(h, Wp_ref[...], preferred_element_type=jnp.float32) + bp_ref[...]

    x_ref[:, 0:F] = tsf_ref[...]
    x_ref[:, F:2 * F] = text_feat
    xcat = x_ref[...]
    logits = jnp.dot(xcat, Wg_ref[...], preferred_element_type=jnp.float32) + bg_ref[...]
    logit_ref[...] = jnp.swapaxes(logits.reshape(BB // 32, 32, E), 1, 2)


def _router_sc_body(lt_hbm, w_hbm, lt_v, w_v):
    wid = lax.axis_index("s") * 2 + lax.axis_index("c")
    rows = B // 32
    base = wid * rows * E
    pltpu.sync_copy(lt_hbm.at[pl.ds(base, rows * E)], lt_v)
    for c in range(rows // 16):
        ls = [lt_v[pl.ds(e * rows + c * 16, 16)] for e in range(E)]
        m = ls[0]
        for e in range(1, E):
            m = jnp.maximum(m, ls[e])
        ex = [jnp.exp(l - m) for l in ls]
        ssum = ex[0]
        for e in range(1, E):
            ssum = ssum + ex[e]
        ps = [v / ssum for v in ex]
        v1 = ps[0]
        for e in range(1, E):
            v1 = jnp.maximum(v1, ps[e])
        i1 = jnp.full((16,), E, jnp.int32)
        for e in reversed(range(E)):
            i1 = jnp.where(ps[e] == v1, e, i1)
        p2 = [jnp.where(i1 == e, -1.0, ps[e]) for e in range(E)]
        v2 = p2[0]
        for e in range(1, E):
            v2 = jnp.maximum(v2, p2[e])
        i2 = jnp.full((16,), E, jnp.int32)
        for e in reversed(range(E)):
            i2 = jnp.where(p2[e] == v2, e, i2)
        denom = v1 + v2
        for e in range(E):
            w_v[pl.ds(e * rows + c * 16, 16)] = jnp.where(
                i1 == e, v1, jnp.where(i2 == e, v2, 0.0)) / denom
    pltpu.sync_copy(w_v, w_hbm.at[pl.ds(base, rows * E)])


def _router_sc(lt_blk):
    rows = B // 32
    w_blk = pl.kernel(
        _router_sc_body,
        mesh=plsc.VectorSubcoreMesh(core_axis_name="c", subcore_axis_name="s"),
        out_type=jax.ShapeDtypeStruct((B * E,), jnp.float32),
        scratch_types=[
            pltpu.VMEM((rows * E,), jnp.float32),
            pltpu.VMEM((rows * E,), jnp.float32),
        ],
    )(lt_blk)
    return w_blk.reshape(32, E, rows)


def _moe_body(x_ref, w_ref, We1_ref, be1_ref, We2_ref, be2_ref,
              Wt12a_ref, bt12a_ref, Wt12b_ref, bt12b_ref,
              out_ref):
    x = x_ref[...]
    wb = w_ref[...]
    w = jnp.swapaxes(wb, 1, 2).reshape(x.shape[0], E)

    xb = x.astype(jnp.bfloat16)
    moe = jnp.zeros((x.shape[0], 2 * F), jnp.float32)
    for e in range(E):
        eh = jax.nn.gelu(jnp.dot(xb, We1_ref[e].astype(jnp.bfloat16),
                                 preferred_element_type=jnp.float32) + be1_ref[e])
        eo = jnp.dot(eh.astype(jnp.bfloat16), We2_ref[e].astype(jnp.bfloat16),
                     preferred_element_type=jnp.float32) + be2_ref[e]
        moe = moe + eo * w[:, e:e + 1]

    tcat = jnp.maximum(jnp.dot(moe, Wt12a_ref[...], preferred_element_type=jnp.float32) + bt12a_ref[...], 0.0)
    out_ref[...] = jnp.dot(tcat, Wt12b_ref[...], preferred_element_type=jnp.float32) + bt12b_ref[...]


def kernel(ts_x, bert_x, params):
    p = params
    ts_tm = jnp.swapaxes(ts_x, 0, 1)
    bert_tm = jnp.swapaxes(bert_x, 0, 1)

    r1 = lambda a: a.reshape(1, -1)
    hd = F // 4
    Wqkv = jnp.concatenate(
        [p['Wq'] * np.float32(1.0 / np.sqrt(hd)), p['Wk'], p['Wv']], axis=1)

    Wxh = jnp.concatenate([p['Wx'], p['Wh']], axis=0)

    BB1A = 256
    hs = pl.pallas_call(
        _conv_lstm_body,
        grid=(B // BB1A,),
        in_specs=[
            pl.BlockSpec((T, BB1A, CIN), lambda i: (0, i, 0)),
            _fullspec((3, CIN, F)), _fullspec((1, F)),
            _fullspec((3, F, F)), _fullspec((1, F)),
            _fullspec((2 * F, H4)), _fullspec((1, H4)),
        ],
        out_specs=pl.BlockSpec((BB1A, T, F), lambda i: (i, 0, 0)),
        out_shape=jax.ShapeDtypeStruct((B, T, F), jnp.float32),
        scratch_shapes=[pltpu.VMEM((T, BB1A, F), jnp.float32)],
    )(ts_tm, p['Wc1'], r1(p['bc1']), p['Wc2'], r1(p['bc2']),
      Wxh, r1(p['bl']))

    BB1 = 128
    ts_feat = pl.pallas_call(
        _mha_ffn_body,
        grid=(B // BB1,),
        in_specs=[
            pl.BlockSpec((BB1, T, F), lambda i: (i, 0, 0)),
            _fullspec((F, 3 * F)), _fullspec((F, F)),
            _fullspec((1, F)), _fullspec((1, F)),
            _fullspec((F, DFF)), _fullspec((1, DFF)),
            _fullspec((DFF, F)), _fullspec((1, F)),
            _fullspec((1, F)), _fullspec((1, F)),
            _fullspec((F, F)), _fullspec((1, F)),
        ],
        out_specs=pl.BlockSpec((BB1, F), lambda i: (i, 0)),
        out_shape=jax.ShapeDtypeStruct((B, F), jnp.float32),
    )(hs, Wqkv, p['Wo'],
      r1(p['ln1g']), r1(p['ln1b']), p['Wf1'], r1(p['bf1']), p['Wf2'], r1(p['bf2']),
      r1(p['ln2g']), r1(p['ln2b']), p['Wts'], r1(p['bts']))

    BB2 = 256
    xcat, logits = pl.pallas_call(
        _gru_branch_body,
        grid=(B // BB2,),
        in_specs=[
            pl.BlockSpec((S, BB2, BF), lambda i: (0, i, 0)),
            pl.BlockSpec((BB2, F), lambda i: (i, 0)),
            _fullspec((BF, G3)), _fullspec((1, G3)), _fullspec((F, G3)),
            _fullspec((F, F)), _fullspec((1, F)),
            _fullspec((2 * F, E)), _fullspec((1, E)),
        ],
        out_specs=[
            pl.BlockSpec((BB2, 2 * F), lambda i: (i, 0)),
            pl.BlockSpec((BB2 // 32, E, 32), lambda i: (i, 0, 0)),
        ],
        out_shape=[
            jax.ShapeDtypeStruct((B, 2 * F), jnp.float32),
            jax.ShapeDtypeStruct((32, E, 32), jnp.float32),
        ],
        scratch_shapes=[pltpu.VMEM((S, BB2, G3), jnp.float32)],
    )(bert_tm, ts_feat, p['Wgx'], r1(p['bgr']), p['Wgh'],
      p['Wp'], r1(p['bp']), p['Wg'], r1(p['bg']))

    w = _router_sc(logits.reshape(-1))

    BB4 = 256
    out = pl.pallas_call(
        _moe_body,
        grid=(B // BB4,),
        in_specs=[
            pl.BlockSpec((BB4, 2 * F), lambda i: (i, 0)),
            pl.BlockSpec((BB4 // 32, E, 32), lambda i: (i, 0, 0)),
            _fullspec((E, 2 * F, DE)), _fullspec((E, DE)),
            _fullspec((E, DE, 2 * F)), _fullspec((E, 2 * F)),
            _fullspec((2 * F, 2 * F)), _fullspec((1, 2 * F)),
            _fullspec((2 * F, 2)), _fullspec((1, 2)),
        ],
        out_specs=pl.BlockSpec((BB4, 2), lambda i: (i, 0)),
        out_shape=jax.ShapeDtypeStruct((B, 2), jnp.float32),
    )(xcat, w, p['We1'], p['be1'], p['We2'], p['be2'],
      jnp.concatenate([p['Wt1a'], p['Wt2a']], axis=1),
      jnp.concatenate([p['bt1a'], p['bt2a']]).reshape(1, -1),
      jnp.concatenate([
          jnp.concatenate([p['Wt1b'], jnp.zeros((F, 1), jnp.float32)], axis=1),
          jnp.concatenate([jnp.zeros((F, 1), jnp.float32), p['Wt2b']], axis=1),
      ], axis=0),
      jnp.concatenate([p['bt1b'], p['bt2b']]).reshape(1, -1))
    return out
